# Initial kernel scaffold; baseline (speedup 1.0000x reference)
import jax
import jax.numpy as jnp
from jax.experimental import pallas as pl


def kernel(x, W1, b1, W2, b2, W3, b3, g1, be1, m1, v1, g2, be2, m2, v2, g3, be3, m3, v3, Wl1, bl1, Wc1, bc1, Wl2, bl2, Wc2, bc2, W4, b4, W5, b5):
    raise NotImplementedError("write your pallas kernel here")


# speedup vs baseline: 1.1702x; 1.1702x over previous
import jax
import jax.numpy as jnp
from jax.experimental import pallas as pl

K_NEIGHBORS = 16


def _knn(x, k):
    inner = jnp.matmul(x, jnp.swapaxes(x, 1, 2))
    xx = jnp.sum(x * x, axis=-1)
    dist = xx[:, :, None] - 2.0 * inner + xx[:, None, :]
    _, idx = jax.lax.top_k(-dist, k)
    return idx


def _index_points(x, idx):
    return jax.vmap(lambda pts, i: pts[i])(x, idx)


def _bn(x, gamma, beta, mean, var, eps=1e-3):
    return gamma * (x - mean) / jnp.sqrt(var + eps) + beta


def _final_mlp_kernel(h_ref, w4_ref, b4_ref, w5_ref, b5_ref, o_ref):
    h = h_ref[...]
    a = jnp.maximum(jnp.dot(h, w4_ref[...], preferred_element_type=jnp.float32) + b4_ref[...], 0.0)
    o_ref[...] = jnp.dot(a, w5_ref[...], preferred_element_type=jnp.float32) + b5_ref[...]


def kernel(x, W1, b1, W2, b2, W3, b3, g1, be1, m1, v1, g2, be2, m2, v2, g3, be3, m3, v3, Wl1, bl1, Wc1, bc1, Wl2, bl2, Wc2, bc2, W4, b4, W5, b5):
    k = K_NEIGHBORS
    B, N = x.shape[0], x.shape[1]
    idx0 = _knn(x, k)
    knn_x = _index_points(x, idx0)
    mean = jnp.mean(knn_x, axis=2, keepdims=True)
    knn_x = knn_x - mean
    cov = jnp.matmul(jnp.swapaxes(knn_x, 2, 3), knn_x)
    cov_flat = cov.reshape(B, N, 9)
    h = jnp.concatenate([x, cov_flat], axis=2)
    h = jax.nn.relu(_bn(h @ W1 + b1, g1, be1, m1, v1))
    h = jax.nn.relu(_bn(h @ W2 + b2, g2, be2, m2, v2))
    h = jax.nn.relu(_bn(h @ W3 + b3, g3, be3, m3, v3))
    idx1 = _knn(h, k)
    gx = _index_points(h, idx1)
    h = jnp.max(gx, axis=2)
    h = h @ Wl1 + bl1
    h = h @ Wc1 + bc1
    h = jax.nn.relu(h)
    idx2 = _knn(h, k)
    gx = _index_points(h, idx2)
    h = jnp.max(gx, axis=2)
    h = h @ Wl2 + bl2
    h = h @ Wc2 + bc2
    h = jnp.max(h, axis=1)
    out = pl.pallas_call(
        _final_mlp_kernel,
        out_shape=jax.ShapeDtypeStruct((B, W5.shape[1]), jnp.float32),
    )(h, W4, b4, W5, b5)
    return out[:, None, :]


# speedup vs baseline: 12.3826x; 10.5817x over previous
import functools

import jax
import jax.numpy as jnp
from jax import lax
from jax.experimental import pallas as pl
from jax.experimental.pallas import tpu as pltpu
from jax.experimental.pallas import tpu_sc as plsc

K = 16
SC_CORES = 2
SC_SUBCORES = 16
NW = SC_CORES * SC_SUBCORES


def _topk_body(f_all_ref, f_blk_ref, idx_ref, *rest, R, N, with_g):
    b = pl.program_id(0)
    xa = f_all_ref[0]
    xb = f_blk_ref[0]
    inner = lax.dot_general(xb.astype(jnp.bfloat16), xa.astype(jnp.bfloat16),
                            (((1,), (1,)), ((), ())),
                            preferred_element_type=jnp.float32)
    na = jnp.transpose(jnp.sum(xa * xa, axis=1, keepdims=True))
    d = na - 2.0 * inner
    col = lax.broadcasted_iota(jnp.int32, (R, N), 1)
    inf = jnp.float32(jnp.inf)
    outs = []
    for _ in range(K):
        m = jnp.min(d, axis=1, keepdims=True)
        am = jnp.min(jnp.where(d == m, col, N), axis=1, keepdims=True)
        outs.append(am)
        d = jnp.where(col == am, inf, d)
    idx_ref[0] = jnp.concatenate(outs, axis=1) + b * N
    if with_g:
        g_ref = rest[0]
        c0, c1, c2 = (xb[:, a:a + 1] for a in range(3))
        left9 = jnp.concatenate([c0, c0, c0, c1, c1, c1, c2, c2, c2], axis=1)
        right9 = jnp.concatenate([c0, c1, c2, c0, c1, c2, c0, c1, c2], axis=1)
        z7 = jnp.zeros((R, 7), jnp.float32)
        g_ref[0] = jnp.concatenate(
            [left9, z7, right9, jnp.zeros((R, 103), jnp.float32)], axis=1)


def _topk_call(f, R, with_g):
    B, N, D = f.shape
    body = functools.partial(_topk_body, R=R, N=N, with_g=with_g)
    out_shape = [jax.ShapeDtypeStruct((B, N, K), jnp.int32)]
    out_specs = [pl.BlockSpec((1, R, K), lambda b, r: (b, r, 0))]
    if with_g:
        out_shape.append(jax.ShapeDtypeStruct((B, N, 128), jnp.float32))
        out_specs.append(pl.BlockSpec((1, R, 128), lambda b, r: (b, r, 0)))
    return pl.pallas_call(
        body,
        grid=(B, N // R),
        in_specs=[
            pl.BlockSpec((1, N, D), lambda b, r: (b, 0, 0)),
            pl.BlockSpec((1, R, D), lambda b, r: (b, r, 0)),
        ],
        out_specs=out_specs,
        out_shape=out_shape,
    )(f, f)


def _mm(a, w):
    return jnp.dot(a.astype(jnp.bfloat16), w.astype(jnp.bfloat16),
                   preferred_element_type=jnp.float32)


def _gather_combine_jnp(table, idx_flat, Dout, op):
    BN = table.shape[0]
    rows = table[idx_flat].reshape(BN, K, table.shape[1])
    if op is jnp.add:
        return jnp.sum(rows, axis=1)[:, :Dout]
    return jnp.max(rows, axis=1)[:, :Dout]


def _gather_cov(table, idx_flat):
    BN = table.shape[0]
    CH = 8
    pw = BN // NW
    mesh = plsc.VectorSubcoreMesh(core_axis_name="c", subcore_axis_name="s")

    @functools.partial(
        pl.kernel,
        mesh=mesh,
        out_type=jax.ShapeDtypeStruct((BN, 16), jnp.float32),
        scratch_types=[
            pltpu.VMEM((CH * K,), jnp.int32),
            pltpu.VMEM((CH * K, 128), jnp.float32),
            pltpu.VMEM((CH, 16), jnp.float32),
            pltpu.SemaphoreType.DMA,
        ],
    )
    def gk(table_hbm, idx_hbm, out_hbm, idx_v, rows_v, out_v, sem):
        wid = lax.axis_index("s") * SC_CORES + lax.axis_index("c")
        base0 = wid * pw

        def rnd_bf16(v):
            t = v * 65537.0
            return t - (t - v)

        def step(i, carry):
            base = base0 + i * CH
            pltpu.sync_copy(idx_hbm.at[pl.ds(base * K, CH * K)], idx_v)
            pltpu.async_copy(table_hbm.at[idx_v], rows_v, sem).wait()
            sll = pl.ds(0, 16)
            slr = pl.ds(16, 16)
            for p in range(CH):
                accl = rows_v[p * K, sll]
                accr = rows_v[p * K, slr]
                for r in range(1, K):
                    accl = accl + rows_v[p * K + r, sll]
                    accr = accr + rows_v[p * K + r, slr]
                meanl = accl * (1.0 / K)
                meanr = accr * (1.0 / K)
                cov = None
                for r in range(K):
                    ql = rnd_bf16(rows_v[p * K + r, sll] - meanl)
                    qr = rnd_bf16(rows_v[p * K + r, slr] - meanr)
                    prod = ql * qr
                    cov = prod if cov is None else cov + prod
                out_v[p, sll] = cov
            pltpu.sync_copy(out_v, out_hbm.at[pl.ds(base, CH)])
            return carry

        lax.fori_loop(0, pw // CH, step, 0)

    return gk(table, idx_flat)


def _gather_cov_jnp(table, idx_flat):
    BN = table.shape[0]
    rows = table[idx_flat].reshape(BN, K, 128)[:, :, :3]
    mean = jnp.mean(rows, axis=1, keepdims=True)
    dx = (rows - mean).astype(jnp.bfloat16).astype(jnp.float32)
    cov = jnp.einsum('pka,pkc->pac', dx, dx)
    out = jnp.zeros((BN, 48), jnp.float32)
    out = out.at[:, 0:3].set(cov[:, 0])
    out = out.at[:, 16:19].set(cov[:, 1])
    out = out.at[:, 32:35].set(cov[:, 2])
    return out


def _gather_combine(table, idx_flat, Dout, op):
    BN = table.shape[0]
    CH = 8
    pw = BN // NW
    mesh = plsc.VectorSubcoreMesh(core_axis_name="c", subcore_axis_name="s")

    @functools.partial(
        pl.kernel,
        mesh=mesh,
        out_type=jax.ShapeDtypeStruct((BN, Dout), jnp.float32),
        scratch_types=[
            pltpu.VMEM((CH * K,), jnp.int32),
            pltpu.VMEM((CH * K, 128), jnp.float32),
            pltpu.VMEM((CH, Dout), jnp.float32),
            pltpu.SemaphoreType.DMA,
        ],
    )
    def gk(table_hbm, idx_hbm, out_hbm, idx_v, rows_v, out_v, sem):
        wid = lax.axis_index("s") * SC_CORES + lax.axis_index("c")
        base0 = wid * pw

        def step(i, carry):
            base = base0 + i * CH
            pltpu.sync_copy(idx_hbm.at[pl.ds(base * K, CH * K)], idx_v)
            pltpu.async_copy(table_hbm.at[idx_v], rows_v, sem).wait()
            for p in range(CH):
                for c in range(Dout // 16):
                    sl = pl.ds(c * 16, 16)
                    acc = rows_v[p * K, sl]
                    for r in range(1, K):
                        acc = op(acc, rows_v[p * K + r, sl])
                    out_v[p, sl] = acc
            pltpu.sync_copy(out_v, out_hbm.at[pl.ds(base, CH)])
            return carry

        lax.fori_loop(0, pw // CH, step, 0)

    return gk(table, idx_flat)


def _c0_body(x_ref, s_ref, w1_ref, w2_ref, w3_ref, sc1_ref, sh1_ref,
             sc2_ref, sh2_ref, sc3_ref, sh3_ref, h_ref):
    x = x_ref[0]
    s = s_ref[0]
    N = x.shape[0]
    h16 = jnp.concatenate(
        [x[:, 0:3], s[:, 0:9], jnp.zeros((N, 4), jnp.float32)], axis=1)
    a1 = jnp.maximum(
        sc1_ref[...] * _mm(h16, w1_ref[...])
        + sh1_ref[...], 0.0)
    a2 = jnp.maximum(
        sc2_ref[...] * _mm(a1, w2_ref[...])
        + sh2_ref[...], 0.0)
    a3 = jnp.maximum(
        sc3_ref[...] * _mm(a2, w3_ref[...])
        + sh3_ref[...], 0.0)
    h_ref[0] = jnp.concatenate([a3, jnp.zeros((N, 64), jnp.float32)], axis=1)


def _c0_call(x8, S, w1p, w2p, w3p, sc1, sh1, sc2, sh2, sc3, sh3):
    B, N = x8.shape[0], x8.shape[1]
    full = lambda a: pl.BlockSpec(a.shape, lambda b: (0,) * a.ndim)
    return pl.pallas_call(
        _c0_body,
        grid=(B,),
        in_specs=[pl.BlockSpec((1, N, 8), lambda b: (b, 0, 0)),
                  pl.BlockSpec((1, N, 16), lambda b: (b, 0, 0)),
                  full(w1p), full(w2p), full(w3p),
                  full(sc1), full(sh1), full(sc2), full(sh2),
                  full(sc3), full(sh3)],
        out_specs=pl.BlockSpec((1, N, 128), lambda b: (b, 0, 0)),
        out_shape=jax.ShapeDtypeStruct((B, N, 128), jnp.float32),
    )(x8, S, w1p, w2p, w3p, sc1, sh1, sc2, sh2, sc3, sh3)


def _c1_body(g_ref, wl_ref, bl_ref, wc_ref, bc_ref, h_ref):
    g = g_ref[0]
    t = _mm(g, wl_ref[...]) \
        + bl_ref[...]
    h_ref[0] = jnp.maximum(
        _mm(t, wc_ref[...])
        + bc_ref[...], 0.0)


def _c1_call(g, wl, bl, wc, bc):
    B, N = g.shape[0], g.shape[1]
    Din, Dout = wl.shape[0], wc.shape[1]
    full = lambda a: pl.BlockSpec(a.shape, lambda b: (0,) * a.ndim)
    return pl.pallas_call(
        _c1_body,
        grid=(B,),
        in_specs=[pl.BlockSpec((1, N, Din), lambda b: (b, 0, 0)),
                  full(wl), full(bl), full(wc), full(bc)],
        out_specs=pl.BlockSpec((1, N, Dout), lambda b: (b, 0, 0)),
        out_shape=jax.ShapeDtypeStruct((B, N, Dout), jnp.float32),
    )(g, wl, bl, wc, bc)


def _c2_body(g_ref, wl_ref, bl_ref, wc_ref, bc_ref, w4_ref, b4_ref,
             w5_ref, b5_ref, o_ref):
    g = g_ref[0]
    t = _mm(g, wl_ref[...]) \
        + bl_ref[...]
    t = _mm(t, wc_ref[...]) \
        + bc_ref[...]
    p = jnp.max(t, axis=0, keepdims=True)
    a = jnp.maximum(
        _mm(p, w4_ref[...])
        + b4_ref[...], 0.0)
    o_ref[0] = _mm(a, w5_ref[...]) \
        + b5_ref[...]


def _c2_call(g, wl, bl, wc, bc, w4, b4, w5, b5):
    B, N, Din = g.shape
    latent = w5.shape[1]
    full = lambda a: pl.BlockSpec(a.shape, lambda b: (0,) * a.ndim)
    return pl.pallas_call(
        _c2_body,
        grid=(B,),
        in_specs=[pl.BlockSpec((1, N, Din), lambda b: (b, 0, 0)),
                  full(wl), full(bl), full(wc), full(bc),
                  full(w4), full(b4), full(w5), full(b5)],
        out_specs=pl.BlockSpec((1, 1, latent), lambda b: (b, 0, 0)),
        out_shape=jax.ShapeDtypeStruct((B, 1, latent), jnp.float32),
    )(g, wl, bl, wc, bc, w4, b4, w5, b5)


_GC0 = _gather_cov
_GC1 = _gather_combine
_GC2 = _gather_combine


def kernel(x, W1, b1, W2, b2, W3, b3, g1, be1, m1, v1, g2, be2, m2, v2,
           g3, be3, m3, v3, Wl1, bl1, Wc1, bc1, Wl2, bl2, Wc2, bc2,
           W4, b4, W5, b5):
    B, N = x.shape[0], x.shape[1]
    BN = B * N
    eps = 1e-3

    def fold(g, be, m, v, bias, pad_to):
        sc = g / jnp.sqrt(v + eps)
        sh = be - m * sc + sc * bias
        pw = pad_to - sc.shape[0]
        sc = jnp.pad(sc, (0, pw))
        sh = jnp.pad(sh, (0, pw))
        return sc[None, :], sh[None, :]

    sc1, sh1 = fold(g1, be1, m1, v1, b1, 16)
    sc2, sh2 = fold(g2, be2, m2, v2, b2, 64)
    sc3, sh3 = fold(g3, be3, m3, v3, b3, 64)
    w1p = jnp.pad(W1, ((0, 4), (0, 4)))
    w2p = jnp.pad(W2, ((0, 4), (0, 0)))

    x8 = jnp.pad(x, ((0, 0), (0, 0), (0, 5)))

    idx0, G = _topk_call(x8, R=256, with_g=True)
    S = _GC0(G.reshape(BN, 128), idx0.reshape(-1))
    h0 = _c0_call(x8, S.reshape(B, N, 16), w1p, w2p, W3,
                  sc1, sh1, sc2, sh2, sc3, sh3)

    (idx1,) = _topk_call(h0, R=256, with_g=False)
    gmax1 = _GC1(h0.reshape(BN, 128), idx1.reshape(-1), 64,
                            jnp.maximum)
    h1 = _c1_call(gmax1.reshape(B, N, 64), Wl1, bl1[None, :],
                  Wc1, bc1[None, :])

    (idx2,) = _topk_call(h1, R=256, with_g=False)
    gmax2 = _GC2(h1.reshape(BN, 128), idx2.reshape(-1), 128,
                            jnp.maximum)
    out = _c2_call(gmax2.reshape(B, N, 128), Wl2, bl2[None, :],
                   Wc2, bc2[None, :], W4, b4[None, :], W5, b5[None, :])
    return out


# speedup vs baseline: 13.2368x; 1.0690x over previous
import functools

import jax
import jax.numpy as jnp
from jax import lax
from jax.experimental import pallas as pl
from jax.experimental.pallas import tpu as pltpu
from jax.experimental.pallas import tpu_sc as plsc

K = 16
SC_CORES = 2
SC_SUBCORES = 16
NW = SC_CORES * SC_SUBCORES


def _topk_body(f_all_ref, f_blk_ref, idx_ref, *rest, R, N, with_g):
    b = pl.program_id(0)
    xa = f_all_ref[0]
    xb = f_blk_ref[0]
    inner = lax.dot_general(xb.astype(jnp.bfloat16), xa.astype(jnp.bfloat16),
                            (((1,), (1,)), ((), ())),
                            preferred_element_type=jnp.float32)
    na = jnp.transpose(jnp.sum(xa * xa, axis=1, keepdims=True))
    d = na - 2.0 * inner
    col = lax.broadcasted_iota(jnp.int32, (R, N), 1)
    inf = jnp.float32(jnp.inf)
    outs = []
    for _ in range(K):
        m = jnp.min(d, axis=1, keepdims=True)
        am = jnp.min(jnp.where(d == m, col, N), axis=1, keepdims=True)
        outs.append(am)
        d = jnp.where(col == am, inf, d)
    idx_ref[0] = jnp.concatenate(outs, axis=1) + b * N
    if with_g:
        g_ref = rest[0]
        c0, c1, c2 = (xb[:, a:a + 1] for a in range(3))
        left9 = jnp.concatenate([c0, c0, c0, c1, c1, c1, c2, c2, c2], axis=1)
        right9 = jnp.concatenate([c0, c1, c2, c0, c1, c2, c0, c1, c2], axis=1)
        z7 = jnp.zeros((R, 7), jnp.float32)
        g_ref[0] = jnp.concatenate(
            [left9, z7, right9, jnp.zeros((R, 103), jnp.float32)], axis=1)


def _topk_call(f, R, with_g):
    B, N, D = f.shape
    body = functools.partial(_topk_body, R=R, N=N, with_g=with_g)
    out_shape = [jax.ShapeDtypeStruct((B, N, K), jnp.int32)]
    out_specs = [pl.BlockSpec((1, R, K), lambda b, r: (b, r, 0))]
    if with_g:
        out_shape.append(jax.ShapeDtypeStruct((B, N, 128), jnp.float32))
        out_specs.append(pl.BlockSpec((1, R, 128), lambda b, r: (b, r, 0)))
    return pl.pallas_call(
        body,
        grid=(B, N // R),
        in_specs=[
            pl.BlockSpec((1, N, D), lambda b, r: (b, 0, 0)),
            pl.BlockSpec((1, R, D), lambda b, r: (b, r, 0)),
        ],
        out_specs=out_specs,
        out_shape=out_shape,
    )(f, f)


def _mm(a, w):
    return jnp.dot(a.astype(jnp.bfloat16), w.astype(jnp.bfloat16),
                   preferred_element_type=jnp.float32)


def _gather_combine_jnp(table, idx_flat, Dout, op):
    BN = table.shape[0]
    rows = table[idx_flat].reshape(BN, K, table.shape[1])
    if op is jnp.add:
        return jnp.sum(rows, axis=1)[:, :Dout]
    return jnp.max(rows, axis=1)[:, :Dout]


def _sc_gather_template(table, idx_flat, Dout, combine):
    BN = table.shape[0]
    CH = 8
    pw = BN // NW
    nch = pw // CH
    mesh = plsc.VectorSubcoreMesh(core_axis_name="c", subcore_axis_name="s")

    @functools.partial(
        pl.kernel,
        mesh=mesh,
        out_type=jax.ShapeDtypeStruct((BN, Dout), jnp.float32),
        scratch_types=[
            pltpu.VMEM((CH * K,), jnp.int32),
            pltpu.VMEM((CH * K,), jnp.int32),
            pltpu.VMEM((CH * K, 128), jnp.float32),
            pltpu.VMEM((CH * K, 128), jnp.float32),
            pltpu.VMEM((CH, Dout), jnp.float32),
            pltpu.SemaphoreType.DMA,
            pltpu.SemaphoreType.DMA,
        ],
    )
    def gk(table_hbm, idx_hbm, out_hbm, idx_a, idx_b, rows_a, rows_b,
           out_v, sem_a, sem_b):
        wid = lax.axis_index("s") * SC_CORES + lax.axis_index("c")
        base0 = wid * pw
        bufs = ((idx_a, rows_a, sem_a), (idx_b, rows_b, sem_b))

        def fetch(c, idxv, rowsv, sem):
            pltpu.sync_copy(idx_hbm.at[pl.ds((base0 + c * CH) * K, CH * K)],
                            idxv)
            pltpu.async_copy(table_hbm.at[idxv], rowsv, sem)

        fetch(0, *bufs[0])
        fetch(1, *bufs[1])

        def step(j, carry):
            for bi in range(2):
                idxv, rowsv, sem = bufs[bi]
                c = j * 2 + bi
                pltpu.make_async_copy(table_hbm.at[idxv], rowsv, sem).wait()
                for p in range(CH):
                    combine(rowsv, out_v, p)
                pltpu.sync_copy(
                    out_v, out_hbm.at[pl.ds(base0 + c * CH, CH)])

                @pl.when(c + 2 < nch)
                def _():
                    fetch(c + 2, idxv, rowsv, sem)
            return carry

        lax.fori_loop(0, nch // 2, step, 0)

    return gk(table, idx_flat)


def _gather_cov(table, idx_flat):
    def rnd_bf16(v):
        t = v * 65537.0
        return t - (t - v)

    def combine(rows_v, out_v, p):
        sll = pl.ds(0, 16)
        slr = pl.ds(16, 16)
        accl = rows_v[p * K, sll]
        accr = rows_v[p * K, slr]
        for r in range(1, K):
            accl = accl + rows_v[p * K + r, sll]
            accr = accr + rows_v[p * K + r, slr]
        meanl = accl * (1.0 / K)
        meanr = accr * (1.0 / K)
        cov = None
        for r in range(K):
            ql = rnd_bf16(rows_v[p * K + r, sll] - meanl)
            qr = rnd_bf16(rows_v[p * K + r, slr] - meanr)
            prod = ql * qr
            cov = prod if cov is None else cov + prod
        out_v[p, sll] = cov

    return _sc_gather_template(table, idx_flat, 16, combine)


def _gather_combine(table, idx_flat, Dout, op):
    def combine(rows_v, out_v, p):
        for c in range(Dout // 16):
            sl = pl.ds(c * 16, 16)
            acc = rows_v[p * K, sl]
            for r in range(1, K):
                acc = op(acc, rows_v[p * K + r, sl])
            out_v[p, sl] = acc

    return _sc_gather_template(table, idx_flat, Dout, combine)


def _c0_body(x_ref, s_ref, w1_ref, w2_ref, w3_ref, sc1_ref, sh1_ref,
             sc2_ref, sh2_ref, sc3_ref, sh3_ref, h_ref):
    x = x_ref[0]
    s = s_ref[0]
    N = x.shape[0]
    h16 = jnp.concatenate(
        [x[:, 0:3], s[:, 0:9], jnp.zeros((N, 4), jnp.float32)], axis=1)
    a1 = jnp.maximum(
        sc1_ref[...] * _mm(h16, w1_ref[...])
        + sh1_ref[...], 0.0)
    a2 = jnp.maximum(
        sc2_ref[...] * _mm(a1, w2_ref[...])
        + sh2_ref[...], 0.0)
    a3 = jnp.maximum(
        sc3_ref[...] * _mm(a2, w3_ref[...])
        + sh3_ref[...], 0.0)
    h_ref[0] = jnp.concatenate([a3, jnp.zeros((N, 64), jnp.float32)], axis=1)


def _c0_call(x8, S, w1p, w2p, w3p, sc1, sh1, sc2, sh2, sc3, sh3):
    B, N = x8.shape[0], x8.shape[1]
    full = lambda a: pl.BlockSpec(a.shape, lambda b: (0,) * a.ndim)
    return pl.pallas_call(
        _c0_body,
        grid=(B,),
        in_specs=[pl.BlockSpec((1, N, 8), lambda b: (b, 0, 0)),
                  pl.BlockSpec((1, N, 16), lambda b: (b, 0, 0)),
                  full(w1p), full(w2p), full(w3p),
                  full(sc1), full(sh1), full(sc2), full(sh2),
                  full(sc3), full(sh3)],
        out_specs=pl.BlockSpec((1, N, 128), lambda b: (b, 0, 0)),
        out_shape=jax.ShapeDtypeStruct((B, N, 128), jnp.float32),
    )(x8, S, w1p, w2p, w3p, sc1, sh1, sc2, sh2, sc3, sh3)


def _c1_body(g_ref, wl_ref, bl_ref, wc_ref, bc_ref, h_ref):
    g = g_ref[0]
    t = _mm(g, wl_ref[...]) \
        + bl_ref[...]
    h_ref[0] = jnp.maximum(
        _mm(t, wc_ref[...])
        + bc_ref[...], 0.0)


def _c1_call(g, wl, bl, wc, bc):
    B, N = g.shape[0], g.shape[1]
    Din, Dout = wl.shape[0], wc.shape[1]
    full = lambda a: pl.BlockSpec(a.shape, lambda b: (0,) * a.ndim)
    return pl.pallas_call(
        _c1_body,
        grid=(B,),
        in_specs=[pl.BlockSpec((1, N, Din), lambda b: (b, 0, 0)),
                  full(wl), full(bl), full(wc), full(bc)],
        out_specs=pl.BlockSpec((1, N, Dout), lambda b: (b, 0, 0)),
        out_shape=jax.ShapeDtypeStruct((B, N, Dout), jnp.float32),
    )(g, wl, bl, wc, bc)


def _c2_body(g_ref, wl_ref, bl_ref, wc_ref, bc_ref, w4_ref, b4_ref,
             w5_ref, b5_ref, o_ref):
    g = g_ref[0]
    t = _mm(g, wl_ref[...]) \
        + bl_ref[...]
    t = _mm(t, wc_ref[...]) \
        + bc_ref[...]
    p = jnp.max(t, axis=0, keepdims=True)
    a = jnp.maximum(
        _mm(p, w4_ref[...])
        + b4_ref[...], 0.0)
    o_ref[0] = _mm(a, w5_ref[...]) \
        + b5_ref[...]


def _c2_call(g, wl, bl, wc, bc, w4, b4, w5, b5):
    B, N, Din = g.shape
    latent = w5.shape[1]
    full = lambda a: pl.BlockSpec(a.shape, lambda b: (0,) * a.ndim)
    return pl.pallas_call(
        _c2_body,
        grid=(B,),
        in_specs=[pl.BlockSpec((1, N, Din), lambda b: (b, 0, 0)),
                  full(wl), full(bl), full(wc), full(bc),
                  full(w4), full(b4), full(w5), full(b5)],
        out_specs=pl.BlockSpec((1, 1, latent), lambda b: (b, 0, 0)),
        out_shape=jax.ShapeDtypeStruct((B, 1, latent), jnp.float32),
    )(g, wl, bl, wc, bc, w4, b4, w5, b5)


_GC0 = _gather_cov
_GC1 = _gather_combine
_GC2 = _gather_combine


def kernel(x, W1, b1, W2, b2, W3, b3, g1, be1, m1, v1, g2, be2, m2, v2,
           g3, be3, m3, v3, Wl1, bl1, Wc1, bc1, Wl2, bl2, Wc2, bc2,
           W4, b4, W5, b5):
    B, N = x.shape[0], x.shape[1]
    BN = B * N
    eps = 1e-3

    def fold(g, be, m, v, bias, pad_to):
        sc = g / jnp.sqrt(v + eps)
        sh = be - m * sc + sc * bias
        pw = pad_to - sc.shape[0]
        sc = jnp.pad(sc, (0, pw))
        sh = jnp.pad(sh, (0, pw))
        return sc[None, :], sh[None, :]

    sc1, sh1 = fold(g1, be1, m1, v1, b1, 16)
    sc2, sh2 = fold(g2, be2, m2, v2, b2, 64)
    sc3, sh3 = fold(g3, be3, m3, v3, b3, 64)
    w1p = jnp.pad(W1, ((0, 4), (0, 4)))
    w2p = jnp.pad(W2, ((0, 4), (0, 0)))

    x8 = jnp.pad(x, ((0, 0), (0, 0), (0, 5)))

    idx0, G = _topk_call(x8, R=256, with_g=True)
    S = _GC0(G.reshape(BN, 128), idx0.reshape(-1))
    h0 = _c0_call(x8, S.reshape(B, N, 16), w1p, w2p, W3,
                  sc1, sh1, sc2, sh2, sc3, sh3)

    (idx1,) = _topk_call(h0, R=256, with_g=False)
    gmax1 = _GC1(h0.reshape(BN, 128), idx1.reshape(-1), 64,
                            jnp.maximum)
    h1 = _c1_call(gmax1.reshape(B, N, 64), Wl1, bl1[None, :],
                  Wc1, bc1[None, :])

    (idx2,) = _topk_call(h1, R=256, with_g=False)
    gmax2 = _GC2(h1.reshape(BN, 128), idx2.reshape(-1), 128,
                            jnp.maximum)
    out = _c2_call(gmax2.reshape(B, N, 128), Wl2, bl2[None, :],
                   Wc2, bc2[None, :], W4, b4[None, :], W5, b5[None, :])
    return out


# speedup vs baseline: 14.4723x; 1.0933x over previous
import functools

import jax
import jax.numpy as jnp
from jax import lax
from jax.experimental import pallas as pl
from jax.experimental.pallas import tpu as pltpu
from jax.experimental.pallas import tpu_sc as plsc

K = 16
SC_CORES = 2
SC_SUBCORES = 16
NW = SC_CORES * SC_SUBCORES


def _topk_body(f_all_ref, f_blk_ref, idx_ref, *rest, R, N, with_g):
    b = pl.program_id(0)
    xa = f_all_ref[0]
    xb = f_blk_ref[0]
    inner = lax.dot_general(xb.astype(jnp.bfloat16), xa.astype(jnp.bfloat16),
                            (((1,), (1,)), ((), ())),
                            preferred_element_type=jnp.float32)
    na = jnp.transpose(jnp.sum(xa * xa, axis=1, keepdims=True))
    d = na - 2.0 * inner
    col = lax.broadcasted_iota(jnp.int32, (R, N), 1)
    inf = jnp.float32(jnp.inf)
    outs = []
    for _ in range(K):
        m = jnp.min(d, axis=1, keepdims=True)
        am = jnp.min(jnp.where(d == m, col, N), axis=1, keepdims=True)
        outs.append(am)
        d = jnp.where(col == am, inf, d)
    idx_ref[0] = jnp.concatenate(outs, axis=1) + b * N
    if with_g:
        g_ref = rest[0]
        c0, c1, c2 = (xb[:, a:a + 1] for a in range(3))
        left9 = jnp.concatenate([c0, c0, c0, c1, c1, c1, c2, c2, c2], axis=1)
        right9 = jnp.concatenate([c0, c1, c2, c0, c1, c2, c0, c1, c2], axis=1)
        z7 = jnp.zeros((R, 7), jnp.float32)
        g_ref[0] = jnp.concatenate(
            [left9, z7, right9, jnp.zeros((R, 103), jnp.float32)], axis=1)


def _topk_call(f, R, with_g):
    B, N, D = f.shape
    body = functools.partial(_topk_body, R=R, N=N, with_g=with_g)
    out_shape = [jax.ShapeDtypeStruct((B, N, K), jnp.int32)]
    out_specs = [pl.BlockSpec((1, R, K), lambda b, r: (b, r, 0))]
    if with_g:
        out_shape.append(jax.ShapeDtypeStruct((B, N, 128), jnp.float32))
        out_specs.append(pl.BlockSpec((1, R, 128), lambda b, r: (b, r, 0)))
    return pl.pallas_call(
        body,
        grid=(B, N // R),
        in_specs=[
            pl.BlockSpec((1, N, D), lambda b, r: (b, 0, 0)),
            pl.BlockSpec((1, R, D), lambda b, r: (b, r, 0)),
        ],
        out_specs=out_specs,
        out_shape=out_shape,
    )(f, f)


def _mm(a, w):
    return jnp.dot(a.astype(jnp.bfloat16), w.astype(jnp.bfloat16),
                   preferred_element_type=jnp.float32)


def _gather_combine_jnp(table, idx_flat, Dout, op):
    BN = table.shape[0]
    rows = table[idx_flat].reshape(BN, K, table.shape[1])
    if op is jnp.add:
        return jnp.sum(rows, axis=1)[:, :Dout]
    return jnp.max(rows, axis=1)[:, :Dout]


def _sc_gather_template(table, idx_flat, Dout, combine):
    BN = table.shape[0]
    CH = 8
    pw = BN // NW
    nch = pw // CH
    mesh = plsc.VectorSubcoreMesh(core_axis_name="c", subcore_axis_name="s")

    @functools.partial(
        pl.kernel,
        mesh=mesh,
        out_type=jax.ShapeDtypeStruct((BN, Dout), jnp.float32),
        scratch_types=[
            pltpu.VMEM((CH * K,), jnp.int32),
            pltpu.VMEM((CH * K,), jnp.int32),
            pltpu.VMEM((CH * K, 128), jnp.float32),
            pltpu.VMEM((CH * K, 128), jnp.float32),
            pltpu.VMEM((CH, Dout), jnp.float32),
            pltpu.SemaphoreType.DMA,
            pltpu.SemaphoreType.DMA,
        ],
    )
    def gk(table_hbm, idx_hbm, out_hbm, idx_a, idx_b, rows_a, rows_b,
           out_v, sem_a, sem_b):
        wid = lax.axis_index("s") * SC_CORES + lax.axis_index("c")
        base0 = wid * pw
        bufs = ((idx_a, rows_a, sem_a), (idx_b, rows_b, sem_b))

        def fetch(c, idxv, rowsv, sem):
            pltpu.sync_copy(idx_hbm.at[pl.ds((base0 + c * CH) * K, CH * K)],
                            idxv)
            pltpu.async_copy(table_hbm.at[idxv], rowsv, sem)

        fetch(0, *bufs[0])
        fetch(1, *bufs[1])

        def step(j, carry):
            for bi in range(2):
                idxv, rowsv, sem = bufs[bi]
                c = j * 2 + bi
                pltpu.make_async_copy(table_hbm.at[idxv], rowsv, sem).wait()
                for p in range(CH):
                    combine(rowsv, out_v, p)
                pltpu.sync_copy(
                    out_v, out_hbm.at[pl.ds(base0 + c * CH, CH)])

                @pl.when(c + 2 < nch)
                def _():
                    fetch(c + 2, idxv, rowsv, sem)
            return carry

        lax.fori_loop(0, nch // 2, step, 0)

    return gk(table, idx_flat)


def _gather_cov(table, idx_flat):
    def rnd_bf16(v):
        t = v * 65537.0
        return t - (t - v)

    def combine(rows_v, out_v, p):
        sll = pl.ds(0, 16)
        slr = pl.ds(16, 16)
        accl = rows_v[p * K, sll]
        accr = rows_v[p * K, slr]
        for r in range(1, K):
            accl = accl + rows_v[p * K + r, sll]
            accr = accr + rows_v[p * K + r, slr]
        meanl = accl * (1.0 / K)
        meanr = accr * (1.0 / K)
        cov = None
        for r in range(K):
            ql = rnd_bf16(rows_v[p * K + r, sll] - meanl)
            qr = rnd_bf16(rows_v[p * K + r, slr] - meanr)
            prod = ql * qr
            cov = prod if cov is None else cov + prod
        out_v[p, sll] = cov

    return _sc_gather_template(table, idx_flat, 16, combine)


def _gather_combine(table, idx_flat, Dout, op):
    def combine(rows_v, out_v, p):
        for c in range(Dout // 16):
            sl = pl.ds(c * 16, 16)
            acc = rows_v[p * K, sl]
            for r in range(1, K):
                acc = op(acc, rows_v[p * K + r, sl])
            out_v[p, sl] = acc

    return _sc_gather_template(table, idx_flat, Dout, combine)


def _c0_body(x_ref, s_ref, w1_ref, w2_ref, w3_ref, sc1_ref, sh1_ref,
             sc2_ref, sh2_ref, sc3_ref, sh3_ref, h_ref):
    x = x_ref[0]
    s = s_ref[0]
    N = x.shape[0]
    h16 = jnp.concatenate(
        [x[:, 0:3], s[:, 0:9], jnp.zeros((N, 4), jnp.float32)], axis=1)
    a1 = jnp.maximum(
        sc1_ref[...] * _mm(h16, w1_ref[...])
        + sh1_ref[...], 0.0)
    a2 = jnp.maximum(
        sc2_ref[...] * _mm(a1, w2_ref[...])
        + sh2_ref[...], 0.0)
    a3 = jnp.maximum(
        sc3_ref[...] * _mm(a2, w3_ref[...])
        + sh3_ref[...], 0.0)
    h_ref[0] = jnp.concatenate([a3, jnp.zeros((N, 64), jnp.float32)], axis=1)


def _c0_call(x8, S, w1p, w2p, w3p, sc1, sh1, sc2, sh2, sc3, sh3):
    B, N = x8.shape[0], x8.shape[1]
    full = lambda a: pl.BlockSpec(a.shape, lambda b: (0,) * a.ndim)
    return pl.pallas_call(
        _c0_body,
        grid=(B,),
        in_specs=[pl.BlockSpec((1, N, 8), lambda b: (b, 0, 0)),
                  pl.BlockSpec((1, N, 16), lambda b: (b, 0, 0)),
                  full(w1p), full(w2p), full(w3p),
                  full(sc1), full(sh1), full(sc2), full(sh2),
                  full(sc3), full(sh3)],
        out_specs=pl.BlockSpec((1, N, 128), lambda b: (b, 0, 0)),
        out_shape=jax.ShapeDtypeStruct((B, N, 128), jnp.float32),
    )(x8, S, w1p, w2p, w3p, sc1, sh1, sc2, sh2, sc3, sh3)


def _c1_body(g_ref, wl_ref, bl_ref, wc_ref, bc_ref, h_ref):
    g = g_ref[0]
    t = _mm(g, wl_ref[...]) \
        + bl_ref[...]
    h_ref[0] = jnp.maximum(
        _mm(t, wc_ref[...])
        + bc_ref[...], 0.0)


def _c1_call(g, wl, bl, wc, bc):
    B, N = g.shape[0], g.shape[1]
    Din, Dout = wl.shape[0], wc.shape[1]
    full = lambda a: pl.BlockSpec(a.shape, lambda b: (0,) * a.ndim)
    return pl.pallas_call(
        _c1_body,
        grid=(B,),
        in_specs=[pl.BlockSpec((1, N, Din), lambda b: (b, 0, 0)),
                  full(wl), full(bl), full(wc), full(bc)],
        out_specs=pl.BlockSpec((1, N, Dout), lambda b: (b, 0, 0)),
        out_shape=jax.ShapeDtypeStruct((B, N, Dout), jnp.float32),
    )(g, wl, bl, wc, bc)


def _c2_body(g_ref, wl_ref, bl_ref, wc_ref, bc_ref, w4_ref, b4_ref,
             w5_ref, b5_ref, o_ref):
    g = g_ref[0]
    t = _mm(g, wl_ref[...]) \
        + bl_ref[...]
    t = _mm(t, wc_ref[...]) \
        + bc_ref[...]
    p = jnp.max(t, axis=0, keepdims=True)
    a = jnp.maximum(
        _mm(p, w4_ref[...])
        + b4_ref[...], 0.0)
    o_ref[0] = _mm(a, w5_ref[...]) \
        + b5_ref[...]


def _c2_call(g, wl, bl, wc, bc, w4, b4, w5, b5):
    B, N, Din = g.shape
    latent = w5.shape[1]
    full = lambda a: pl.BlockSpec(a.shape, lambda b: (0,) * a.ndim)
    return pl.pallas_call(
        _c2_body,
        grid=(B,),
        in_specs=[pl.BlockSpec((1, N, Din), lambda b: (b, 0, 0)),
                  full(wl), full(bl), full(wc), full(bc),
                  full(w4), full(b4), full(w5), full(b5)],
        out_specs=pl.BlockSpec((1, 1, latent), lambda b: (b, 0, 0)),
        out_shape=jax.ShapeDtypeStruct((B, 1, latent), jnp.float32),
    )(g, wl, bl, wc, bc, w4, b4, w5, b5)


_GC0 = _gather_cov
_GC1 = _gather_combine
_GC2 = _gather_combine


def kernel(x, W1, b1, W2, b2, W3, b3, g1, be1, m1, v1, g2, be2, m2, v2,
           g3, be3, m3, v3, Wl1, bl1, Wc1, bc1, Wl2, bl2, Wc2, bc2,
           W4, b4, W5, b5):
    B, N = x.shape[0], x.shape[1]
    BN = B * N
    eps = 1e-3

    def fold(g, be, m, v, bias, pad_to):
        sc = g / jnp.sqrt(v + eps)
        sh = be - m * sc + sc * bias
        pw = pad_to - sc.shape[0]
        sc = jnp.pad(sc, (0, pw))
        sh = jnp.pad(sh, (0, pw))
        return sc[None, :], sh[None, :]

    sc1, sh1 = fold(g1, be1, m1, v1, b1, 16)
    sc2, sh2 = fold(g2, be2, m2, v2, b2, 64)
    sc3, sh3 = fold(g3, be3, m3, v3, b3, 64)
    w1p = jnp.pad(W1, ((0, 4), (0, 4)))
    w2p = jnp.pad(W2, ((0, 4), (0, 0)))

    x8 = jnp.pad(x, ((0, 0), (0, 0), (0, 5)))

    idx0, G = _topk_call(x8, R=512, with_g=True)
    S = _GC0(G.reshape(BN, 128), idx0.reshape(-1))
    h0 = _c0_call(x8, S.reshape(B, N, 16), w1p, w2p, W3,
                  sc1, sh1, sc2, sh2, sc3, sh3)

    (idx1,) = _topk_call(h0, R=512, with_g=False)
    gmax1 = _GC1(h0.reshape(BN, 128), idx1.reshape(-1), 64,
                            jnp.maximum)
    h1 = _c1_call(gmax1.reshape(B, N, 64), Wl1, bl1[None, :],
                  Wc1, bc1[None, :])

    (idx2,) = _topk_call(h1, R=512, with_g=False)
    gmax2 = _GC2(h1.reshape(BN, 128), idx2.reshape(-1), 128,
                            jnp.maximum)
    out = _c2_call(gmax2.reshape(B, N, 128), Wl2, bl2[None, :],
                   Wc2, bc2[None, :], W4, b4[None, :], W5, b5[None, :])
    return out
